# Initial kernel scaffold; baseline (speedup 1.0000x reference)
import jax
import jax.numpy as jnp
from jax.experimental import pallas as pl


def kernel(g1_maccs_x, g1_ecfp_x, g1_pubchem_x, g2_maccs_x, g2_ecfp_x, g2_pubchem_x, gu_maccs_x, gu_ecfp_x, gu_pubchem_x, edge_index_g1, edge_index_g2, edge_index_gu, batch_g1, batch_g2, batch_gu, params):
    raise NotImplementedError("write your pallas kernel here")


# speedup vs baseline: 11.2290x; 11.2290x over previous
import functools
import math

import jax
import jax.numpy as jnp
from jax import lax
from jax.experimental import pallas as pl
from jax.experimental.pallas import tpu as pltpu
from jax.experimental.pallas import tpu_sc as plsc

N_NODES = 10000
N_EDGES = 160000
N_GRAPHS = 128
BN_EPS = 1e-5

NC, NS = 2, 16
CHUNK = 128
CPW = 40
EDGES_PAD = NC * NS * CPW * CHUNK
ROWS_PAD = 10240

BLK = 1000
NBLK = N_NODES // BLK

_f32 = jnp.float32


def _proj_body(xm, xe, xp, wm, we, wp, out):
    dn = (((1,), (1,)), ((), ()))
    ym = lax.dot_general(xm[...], wm[...], dn, preferred_element_type=_f32)
    ye = lax.dot_general(xe[...], we[...], dn, preferred_element_type=_f32)
    yp = lax.dot_general(xp[...], wp[...], dn, preferred_element_type=_f32)
    y = jnp.concatenate([ym, ye, yp], axis=1)
    out[0] = y[:, :96]
    out[1] = y[:, 96:]


def _proj(xm, xe, xp, wm, we, wp):
    return pl.pallas_call(
        _proj_body,
        grid=(NBLK,),
        in_specs=[
            pl.BlockSpec((BLK, xm.shape[1]), lambda i: (i, 0)),
            pl.BlockSpec((BLK, xe.shape[1]), lambda i: (i, 0)),
            pl.BlockSpec((BLK, xp.shape[1]), lambda i: (i, 0)),
            pl.BlockSpec(wm.shape, lambda i: (0, 0)),
            pl.BlockSpec(we.shape, lambda i: (0, 0)),
            pl.BlockSpec(wp.shape, lambda i: (0, 0)),
        ],
        out_specs=pl.BlockSpec((NC, BLK, 96), lambda i: (0, i, 0)),
        out_shape=jax.ShapeDtypeStruct((NC, N_NODES, 96), _f32),
    )(xm, xe, xp, wm, we, wp)


def _make_sc_segsum(D, feature_split):
    mesh = plsc.VectorSubcoreMesh(
        core_axis_name="c", subcore_axis_name="s", num_cores=NC, num_subcores=NS
    )
    cpw = EDGES_PAD // (NS if feature_split else NC * NS) // CHUNK
    zrows = ROWS_PAD // NS

    @functools.partial(
        pl.kernel,
        mesh=mesh,
        out_type=jax.ShapeDtypeStruct((NC, N_NODES, D), _f32),
        scratch_types=[
            pltpu.VMEM((cpw, CHUNK), jnp.int32),
            pltpu.VMEM((cpw, CHUNK), jnp.int32),
            pltpu.VMEM((CHUNK, D), _f32),
            pltpu.VMEM_SHARED((ROWS_PAD, D), _f32),
            pltpu.SemaphoreType.DMA,
        ],
        compiler_params=pltpu.CompilerParams(use_tc_tiling_on_sc=False),
    )
    def segsum(src_hbm, dst_hbm, table_hbm, zeros_hbm, out_hbm,
               src_v, dst_v, rows_v, acc_sh, sem):
        cid = lax.axis_index("c")
        sid = lax.axis_index("s")
        wid = sid if feature_split else cid * NS + sid

        pltpu.sync_copy(zeros_hbm, rows_v)
        for z in range(zrows // CHUNK):
            pltpu.sync_copy(rows_v, acc_sh.at[pl.ds(sid * zrows + z * CHUNK, CHUNK)])
        plsc.subcore_barrier()

        pltpu.sync_copy(src_hbm.at[pl.ds(wid * cpw, cpw)], src_v)
        pltpu.sync_copy(dst_hbm.at[pl.ds(wid * cpw, cpw)], dst_v)

        def chunk_body(j, carry):
            if feature_split:
                src_ref = table_hbm.at[cid].at[src_v.at[j]]
            else:
                src_ref = table_hbm.at[src_v.at[j]]
            pltpu.async_copy(src_ref, rows_v, sem).wait()
            pltpu.sync_copy(rows_v, acc_sh.at[dst_v.at[j]], add=True)
            return carry

        lax.fori_loop(0, cpw, chunk_body, 0)
        plsc.subcore_barrier()

        @pl.when(sid < NS - 1)
        def _():
            pltpu.sync_copy(
                acc_sh.at[pl.ds(sid * 640, 640)],
                out_hbm.at[cid].at[pl.ds(sid * 640, 640)],
            )

        @pl.when(sid == NS - 1)
        def _():
            pltpu.sync_copy(
                acc_sh.at[pl.ds(9600, 400)],
                out_hbm.at[cid].at[pl.ds(9600, 400)],
            )

    return segsum


_sc_segsum_96 = _make_sc_segsum(96, True)
_sc_segsum_24 = _make_sc_segsum(24, False)


def _mid_body(y, agg, b1, w2, b2, sc1, be1, w3, out):
    dn = (((1,), (1,)), ((), ()))
    n1 = jnp.concatenate([y[0] + agg[0], y[1] + agg[1]], axis=1)
    n1 = jnp.maximum(n1 + b1[...], 0.0)
    outs = []
    for f in range(3):
        h = lax.dot_general(n1[:, 64 * f:64 * f + 64], w2[f], dn,
                            preferred_element_type=_f32) + b2[f]
        h = jnp.maximum(h * sc1[f] + be1[f], 0.0)
        outs.append(lax.dot_general(h, w3[f], dn, preferred_element_type=_f32))
    out[...] = jnp.concatenate(outs, axis=1)


def _mid(y, agg, b1, w2, b2, sc1, be1, w3):
    return pl.pallas_call(
        _mid_body,
        grid=(NBLK,),
        in_specs=[
            pl.BlockSpec((NC, BLK, 96), lambda i: (0, i, 0)),
            pl.BlockSpec((NC, BLK, 96), lambda i: (0, i, 0)),
            pl.BlockSpec((1, 192), lambda i: (0, 0)),
            pl.BlockSpec((3, 64, 64), lambda i: (0, 0, 0)),
            pl.BlockSpec((3, 1, 64), lambda i: (0, 0, 0)),
            pl.BlockSpec((3, 1, 64), lambda i: (0, 0, 0)),
            pl.BlockSpec((3, 1, 64), lambda i: (0, 0, 0)),
            pl.BlockSpec((3, 8, 64), lambda i: (0, 0, 0)),
        ],
        out_specs=pl.BlockSpec((BLK, 24), lambda i: (i, 0)),
        out_shape=jax.ShapeDtypeStruct((N_NODES, 24), _f32),
    )(y, agg, b1, w2, b2, sc1, be1, w3)


def _tail_body(u, agg, batch, b3, w4, b4, sc2, be2,
               attw, attb, fc1w, fc1b, fc1s, fc1e, fc2w, fc2b, fc2s, fc2e,
               sim_o, comp_o, embu_o, acc):
    i = pl.program_id(0)
    dn = (((1,), (1,)), ((), ()))
    z = jnp.maximum(u[...] + agg[0] + agg[1] + b3[...], 0.0)
    cols = []
    for f in range(3):
        h = lax.dot_general(z[:, 8 * f:8 * f + 8], w4[f], dn,
                            preferred_element_type=_f32) + b4[f]
        cols.append(jnp.maximum(h * sc2[f] + be2[f], 0.0))
    hc = jnp.concatenate(cols + [jnp.ones((BLK, 1), _f32)], axis=1)

    bb = batch[0]
    iota_g = lax.broadcasted_iota(jnp.int32, (N_GRAPHS, BLK), 0)
    onehot_t = (jnp.broadcast_to(bb, (N_GRAPHS, BLK)) == iota_g).astype(_f32)
    part = lax.dot_general(onehot_t, hc, (((1,), (0,)), ((), ())),
                           preferred_element_type=_f32)

    @pl.when(i == 0)
    def _():
        acc[...] = part

    @pl.when(i > 0)
    def _():
        acc[...] = acc[...] + part

    @pl.when(i == NBLK - 1)
    def _():
        total = acc[...]
        pooled = total[:, :24] / jnp.maximum(total[:, 24:25], 1.0)
        s = lax.dot_general(pooled, attw[...], (((1,), (0,)), ((), ())),
                            preferred_element_type=_f32) + attb[...]
        m = jnp.max(s, axis=1, keepdims=True)
        e = jnp.exp(s - m)
        w = e / jnp.sum(e, axis=1, keepdims=True)
        aggv = (pooled[:, 0:8] * w[:, 0:1] + pooled[:, 8:16] * w[:, 1:2]
                + pooled[:, 16:24] * w[:, 2:3])
        comb = jnp.concatenate([aggv, aggv], axis=1)
        l1 = lax.dot_general(comb, fc1w[...], dn, preferred_element_type=_f32) + fc1b[...]
        l1 = l1 * fc1s[...] + fc1e[...]
        sim_o[...] = (1.0 / (1.0 + jnp.exp(-l1)))[:, 0:1]
        l2 = lax.dot_general(comb, fc2w[...], dn, preferred_element_type=_f32) + fc2b[...]
        l2 = l2 * fc2s[...] + fc2e[...]
        comp_o[...] = jnp.maximum(l2, 0.0)
        embu_o[...] = aggv


def _tail(u, agg, batch3d, b3, w4, b4, sc2, be2,
          attw, attb, fc1w, fc1b, fc1s, fc1e, fc2w, fc2b, fc2s, fc2e):
    full = lambda shp: pl.BlockSpec(shp, lambda i: tuple(0 for _ in shp))
    return pl.pallas_call(
        _tail_body,
        grid=(NBLK,),
        in_specs=[
            pl.BlockSpec((BLK, 24), lambda i: (i, 0)),
            pl.BlockSpec((NC, BLK, 24), lambda i: (0, i, 0)),
            pl.BlockSpec((1, 1, BLK), lambda i: (i, 0, 0)),
            full((1, 24)),
            full((3, 8, 8)), full((3, 1, 8)), full((3, 1, 8)), full((3, 1, 8)),
            full((24, 8)), full((1, 8)),
            full((8, 16)), full((1, 8)), full((1, 8)), full((1, 8)),
            full((8, 16)), full((1, 8)), full((1, 8)), full((1, 8)),
        ],
        out_specs=[
            pl.BlockSpec((N_GRAPHS, 1), lambda i: (0, 0)),
            pl.BlockSpec((N_GRAPHS, 8), lambda i: (0, 0)),
            pl.BlockSpec((N_GRAPHS, 8), lambda i: (0, 0)),
        ],
        out_shape=[
            jax.ShapeDtypeStruct((N_GRAPHS, 1), _f32),
            jax.ShapeDtypeStruct((N_GRAPHS, 8), _f32),
            jax.ShapeDtypeStruct((N_GRAPHS, 8), _f32),
        ],
        scratch_shapes=[pltpu.VMEM((N_GRAPHS, 25), _f32)],
    )(u, agg, batch3d, b3, w4, b4, sc2, be2,
      attw, attb, fc1w, fc1b, fc1s, fc1e, fc2w, fc2b, fc2s, fc2e)


def kernel(g1_maccs_x, g1_ecfp_x, g1_pubchem_x, g2_maccs_x, g2_ecfp_x,
           g2_pubchem_x, gu_maccs_x, gu_ecfp_x, gu_pubchem_x,
           edge_index_g1, edge_index_g2, edge_index_gu,
           batch_g1, batch_g2, batch_gu, params):
    p = params
    pm, pe, pq, t = p['maccs'], p['ecfp'], p['pubchem'], p['top']
    inv = 1.0 / math.sqrt(1.0 + BN_EPS)

    pad = EDGES_PAD - N_EDGES
    src = jnp.concatenate([edge_index_g1[0], jnp.zeros((pad,), jnp.int32)])
    dst = jnp.concatenate([edge_index_g1[1], jnp.full((pad,), N_NODES, jnp.int32)])
    src2d = src.reshape(NC * NS * CPW, CHUNK)
    dst2d = dst.reshape(NC * NS * CPW, CHUNK)

    y = _proj(g1_maccs_x, g1_ecfp_x, g1_pubchem_x, pm['W1'], pe['W1'], pq['W1'])

    agg1 = _sc_segsum_96(src2d, dst2d, y, jnp.zeros((CHUNK, 96), _f32))

    b1 = jnp.concatenate([pm['b1'], pe['b1'], pq['b1']]).reshape(1, 192)
    w2 = jnp.stack([pm['W2'], pe['W2'], pq['W2']])
    b2 = jnp.stack([pm['b2'], pe['b2'], pq['b2']]).reshape(3, 1, 64)
    sc1 = jnp.stack([pm['bn1_g'], pe['bn1_g'], pq['bn1_g']]).reshape(3, 1, 64) * inv
    be1 = jnp.stack([pm['bn1_b'], pe['bn1_b'], pq['bn1_b']]).reshape(3, 1, 64)
    w3 = jnp.stack([pm['W3'], pe['W3'], pq['W3']])
    u = _mid(y, agg1, b1, w2, b2, sc1, be1, w3)

    agg2 = _sc_segsum_24(src2d, dst2d, u, jnp.zeros((CHUNK, 24), _f32))

    b3 = jnp.concatenate([pm['b3'], pe['b3'], pq['b3']]).reshape(1, 24)
    w4 = jnp.stack([pm['W4'], pe['W4'], pq['W4']])
    b4 = jnp.stack([pm['b4'], pe['b4'], pq['b4']]).reshape(3, 1, 8)
    sc2 = jnp.stack([pm['bn2_g'], pe['bn2_g'], pq['bn2_g']]).reshape(3, 1, 8) * inv
    be2 = jnp.stack([pm['bn2_b'], pe['bn2_b'], pq['bn2_b']]).reshape(3, 1, 8)
    batch3d = batch_g1.reshape(NBLK, 1, BLK)

    aw = t['att_W'].reshape(8)
    z8 = jnp.zeros((8,), _f32)
    attw3 = jnp.stack(
        [jnp.concatenate([aw, z8, z8]), jnp.concatenate([z8, aw, z8]),
         jnp.concatenate([z8, z8, aw])] + [jnp.zeros((24,), _f32)] * 5, axis=1)
    attb8 = jnp.concatenate(
        [jnp.broadcast_to(t['att_b'].reshape(1, 1), (1, 3)),
         jnp.full((1, 5), -1e30, _f32)], axis=1)
    fc1w8 = jnp.concatenate([t['fc1_W'], jnp.zeros((7, 16), _f32)], axis=0)
    bcast8 = lambda v: jnp.broadcast_to(v.reshape(1, 1), (1, 8))

    sim, comp, embu = _tail(
        u, agg2, batch3d, b3, w4, b4, sc2, be2,
        attw3, attb8,
        fc1w8, bcast8(t['fc1_b']),
        bcast8(t['fc1_bn_g'] * inv), bcast8(t['fc1_bn_b']),
        t['fc2_W'], t['fc2_b'].reshape(1, 8),
        (t['fc2_bn_g'] * inv).reshape(1, 8), t['fc2_bn_b'].reshape(1, 8),
    )
    return (sim, comp, embu)


# speedup vs baseline: 13.3855x; 1.1920x over previous
import functools
import math

import jax
import jax.numpy as jnp
from jax import lax
from jax.experimental import pallas as pl
from jax.experimental.pallas import tpu as pltpu
from jax.experimental.pallas import tpu_sc as plsc

N_NODES = 10000
N_EDGES = 160000
N_GRAPHS = 128
BN_EPS = 1e-5

NC, NS = 2, 16
CHUNK = 128
CPW = 40
EDGES_PAD = NC * NS * CPW * CHUNK
NBUF = 4
ROWS_PAD = 10240

BLK = 1000
NBLK = N_NODES // BLK

_f32 = jnp.float32


def _proj_body(xm, xe, xp, wm, we, wp, out):
    dn = (((1,), (1,)), ((), ()))
    ym = lax.dot_general(xm[...], wm[...], dn, preferred_element_type=_f32)
    ye = lax.dot_general(xe[...], we[...], dn, preferred_element_type=_f32)
    yp = lax.dot_general(xp[...], wp[...], dn, preferred_element_type=_f32)
    y = jnp.concatenate([ym, ye, yp], axis=1)
    out[0] = y[:, :96]
    out[1] = y[:, 96:]


def _proj(xm, xe, xp, wm, we, wp):
    return pl.pallas_call(
        _proj_body,
        grid=(NBLK,),
        in_specs=[
            pl.BlockSpec((BLK, xm.shape[1]), lambda i: (i, 0)),
            pl.BlockSpec((BLK, xe.shape[1]), lambda i: (i, 0)),
            pl.BlockSpec((BLK, xp.shape[1]), lambda i: (i, 0)),
            pl.BlockSpec(wm.shape, lambda i: (0, 0)),
            pl.BlockSpec(we.shape, lambda i: (0, 0)),
            pl.BlockSpec(wp.shape, lambda i: (0, 0)),
        ],
        out_specs=pl.BlockSpec((NC, BLK, 96), lambda i: (0, i, 0)),
        out_shape=jax.ShapeDtypeStruct((NC, N_NODES, 96), _f32),
    )(xm, xe, xp, wm, we, wp)


def _make_sc_segsum(D, feature_split):
    mesh = plsc.VectorSubcoreMesh(
        core_axis_name="c", subcore_axis_name="s", num_cores=NC, num_subcores=NS
    )
    cpw = EDGES_PAD // (NS if feature_split else NC * NS) // CHUNK
    zrows = ROWS_PAD // NS

    @functools.partial(
        pl.kernel,
        mesh=mesh,
        out_type=jax.ShapeDtypeStruct((NC, N_NODES, D), _f32),
        scratch_types=[
            pltpu.VMEM((cpw, CHUNK), jnp.int32),
            pltpu.VMEM((cpw, CHUNK), jnp.int32),
            pltpu.VMEM((NBUF, CHUNK, D), _f32),
            pltpu.VMEM_SHARED((ROWS_PAD, D), _f32),
            pltpu.SemaphoreType.DMA,
            pltpu.SemaphoreType.DMA,
            pltpu.SemaphoreType.DMA,
            pltpu.SemaphoreType.DMA,
        ],
        compiler_params=pltpu.CompilerParams(use_tc_tiling_on_sc=False),
    )
    def segsum(src_hbm, dst_hbm, table_hbm, zeros_hbm, out_hbm,
               src_v, dst_v, rows_v, acc_sh, sem0, sem1, sem2, sem3):
        sems = [sem0, sem1, sem2, sem3]
        cid = lax.axis_index("c")
        sid = lax.axis_index("s")
        wid = sid if feature_split else cid * NS + sid

        pltpu.sync_copy(zeros_hbm, rows_v.at[0])
        for z in range(zrows // CHUNK):
            pltpu.sync_copy(rows_v.at[0],
                            acc_sh.at[pl.ds(sid * zrows + z * CHUNK, CHUNK)])
        plsc.subcore_barrier()

        pltpu.sync_copy(src_hbm.at[pl.ds(wid * cpw, cpw)], src_v)
        pltpu.sync_copy(dst_hbm.at[pl.ds(wid * cpw, cpw)], dst_v)

        def gref(j):
            if feature_split:
                return table_hbm.at[cid].at[src_v.at[j]]
            return table_hbm.at[src_v.at[j]]

        for b in range(NBUF):
            pltpu.async_copy(gref(b), rows_v.at[b], sems[b])

        def group_body(g, carry):
            for b in range(NBUF):
                j = g * NBUF + b
                pltpu.make_async_copy(gref(j), rows_v.at[b], sems[b]).wait()
                pltpu.sync_copy(rows_v.at[b], acc_sh.at[dst_v.at[j]], add=True)

                @pl.when(j + NBUF < cpw)
                def _():
                    pltpu.async_copy(gref(j + NBUF), rows_v.at[b], sems[b])
            return carry

        lax.fori_loop(0, cpw // NBUF, group_body, 0)
        plsc.subcore_barrier()

        @pl.when(sid < NS - 1)
        def _():
            pltpu.sync_copy(
                acc_sh.at[pl.ds(sid * 640, 640)],
                out_hbm.at[cid].at[pl.ds(sid * 640, 640)],
            )

        @pl.when(sid == NS - 1)
        def _():
            pltpu.sync_copy(
                acc_sh.at[pl.ds(9600, 400)],
                out_hbm.at[cid].at[pl.ds(9600, 400)],
            )

    return segsum


_sc_segsum_96 = _make_sc_segsum(96, True)
_sc_segsum_24 = _make_sc_segsum(24, False)


def _mid_body(y, agg, b1, w2, b2, sc1, be1, w3, out):
    dn = (((1,), (1,)), ((), ()))
    n1 = jnp.concatenate([y[0] + agg[0], y[1] + agg[1]], axis=1)
    n1 = jnp.maximum(n1 + b1[...], 0.0)
    outs = []
    for f in range(3):
        h = lax.dot_general(n1[:, 64 * f:64 * f + 64], w2[f], dn,
                            preferred_element_type=_f32) + b2[f]
        h = jnp.maximum(h * sc1[f] + be1[f], 0.0)
        outs.append(lax.dot_general(h, w3[f], dn, preferred_element_type=_f32))
    out[...] = jnp.concatenate(outs, axis=1)


def _mid(y, agg, b1, w2, b2, sc1, be1, w3):
    return pl.pallas_call(
        _mid_body,
        grid=(NBLK,),
        in_specs=[
            pl.BlockSpec((NC, BLK, 96), lambda i: (0, i, 0)),
            pl.BlockSpec((NC, BLK, 96), lambda i: (0, i, 0)),
            pl.BlockSpec((1, 192), lambda i: (0, 0)),
            pl.BlockSpec((3, 64, 64), lambda i: (0, 0, 0)),
            pl.BlockSpec((3, 1, 64), lambda i: (0, 0, 0)),
            pl.BlockSpec((3, 1, 64), lambda i: (0, 0, 0)),
            pl.BlockSpec((3, 1, 64), lambda i: (0, 0, 0)),
            pl.BlockSpec((3, 8, 64), lambda i: (0, 0, 0)),
        ],
        out_specs=pl.BlockSpec((BLK, 24), lambda i: (i, 0)),
        out_shape=jax.ShapeDtypeStruct((N_NODES, 24), _f32),
    )(y, agg, b1, w2, b2, sc1, be1, w3)


def _tail_body(u, agg, batch, b3, w4, b4, sc2, be2,
               attw, attb, fc1w, fc1b, fc1s, fc1e, fc2w, fc2b, fc2s, fc2e,
               sim_o, comp_o, embu_o, acc):
    i = pl.program_id(0)
    dn = (((1,), (1,)), ((), ()))
    z = jnp.maximum(u[...] + agg[0] + agg[1] + b3[...], 0.0)
    cols = []
    for f in range(3):
        h = lax.dot_general(z[:, 8 * f:8 * f + 8], w4[f], dn,
                            preferred_element_type=_f32) + b4[f]
        cols.append(jnp.maximum(h * sc2[f] + be2[f], 0.0))
    hc = jnp.concatenate(cols + [jnp.ones((BLK, 1), _f32)], axis=1)

    bb = batch[0]
    iota_g = lax.broadcasted_iota(jnp.int32, (N_GRAPHS, BLK), 0)
    onehot_t = (jnp.broadcast_to(bb, (N_GRAPHS, BLK)) == iota_g).astype(_f32)
    part = lax.dot_general(onehot_t, hc, (((1,), (0,)), ((), ())),
                           preferred_element_type=_f32)

    @pl.when(i == 0)
    def _():
        acc[...] = part

    @pl.when(i > 0)
    def _():
        acc[...] = acc[...] + part

    @pl.when(i == NBLK - 1)
    def _():
        total = acc[...]
        pooled = total[:, :24] / jnp.maximum(total[:, 24:25], 1.0)
        s = lax.dot_general(pooled, attw[...], (((1,), (0,)), ((), ())),
                            preferred_element_type=_f32) + attb[...]
        m = jnp.max(s, axis=1, keepdims=True)
        e = jnp.exp(s - m)
        w = e / jnp.sum(e, axis=1, keepdims=True)
        aggv = (pooled[:, 0:8] * w[:, 0:1] + pooled[:, 8:16] * w[:, 1:2]
                + pooled[:, 16:24] * w[:, 2:3])
        comb = jnp.concatenate([aggv, aggv], axis=1)
        l1 = lax.dot_general(comb, fc1w[...], dn, preferred_element_type=_f32) + fc1b[...]
        l1 = l1 * fc1s[...] + fc1e[...]
        sim_o[...] = (1.0 / (1.0 + jnp.exp(-l1)))[:, 0:1]
        l2 = lax.dot_general(comb, fc2w[...], dn, preferred_element_type=_f32) + fc2b[...]
        l2 = l2 * fc2s[...] + fc2e[...]
        comp_o[...] = jnp.maximum(l2, 0.0)
        embu_o[...] = aggv


def _tail(u, agg, batch3d, b3, w4, b4, sc2, be2,
          attw, attb, fc1w, fc1b, fc1s, fc1e, fc2w, fc2b, fc2s, fc2e):
    full = lambda shp: pl.BlockSpec(shp, lambda i: tuple(0 for _ in shp))
    return pl.pallas_call(
        _tail_body,
        grid=(NBLK,),
        in_specs=[
            pl.BlockSpec((BLK, 24), lambda i: (i, 0)),
            pl.BlockSpec((NC, BLK, 24), lambda i: (0, i, 0)),
            pl.BlockSpec((1, 1, BLK), lambda i: (i, 0, 0)),
            full((1, 24)),
            full((3, 8, 8)), full((3, 1, 8)), full((3, 1, 8)), full((3, 1, 8)),
            full((24, 8)), full((1, 8)),
            full((8, 16)), full((1, 8)), full((1, 8)), full((1, 8)),
            full((8, 16)), full((1, 8)), full((1, 8)), full((1, 8)),
        ],
        out_specs=[
            pl.BlockSpec((N_GRAPHS, 1), lambda i: (0, 0)),
            pl.BlockSpec((N_GRAPHS, 8), lambda i: (0, 0)),
            pl.BlockSpec((N_GRAPHS, 8), lambda i: (0, 0)),
        ],
        out_shape=[
            jax.ShapeDtypeStruct((N_GRAPHS, 1), _f32),
            jax.ShapeDtypeStruct((N_GRAPHS, 8), _f32),
            jax.ShapeDtypeStruct((N_GRAPHS, 8), _f32),
        ],
        scratch_shapes=[pltpu.VMEM((N_GRAPHS, 25), _f32)],
    )(u, agg, batch3d, b3, w4, b4, sc2, be2,
      attw, attb, fc1w, fc1b, fc1s, fc1e, fc2w, fc2b, fc2s, fc2e)


def kernel(g1_maccs_x, g1_ecfp_x, g1_pubchem_x, g2_maccs_x, g2_ecfp_x,
           g2_pubchem_x, gu_maccs_x, gu_ecfp_x, gu_pubchem_x,
           edge_index_g1, edge_index_g2, edge_index_gu,
           batch_g1, batch_g2, batch_gu, params):
    p = params
    pm, pe, pq, t = p['maccs'], p['ecfp'], p['pubchem'], p['top']
    inv = 1.0 / math.sqrt(1.0 + BN_EPS)

    pad = EDGES_PAD - N_EDGES
    src = jnp.concatenate([edge_index_g1[0], jnp.zeros((pad,), jnp.int32)])
    dst = jnp.concatenate([edge_index_g1[1], jnp.full((pad,), N_NODES, jnp.int32)])
    src2d = src.reshape(NC * NS * CPW, CHUNK)
    dst2d = dst.reshape(NC * NS * CPW, CHUNK)

    y = _proj(g1_maccs_x, g1_ecfp_x, g1_pubchem_x, pm['W1'], pe['W1'], pq['W1'])

    agg1 = _sc_segsum_96(src2d, dst2d, y, jnp.zeros((CHUNK, 96), _f32))

    b1 = jnp.concatenate([pm['b1'], pe['b1'], pq['b1']]).reshape(1, 192)
    w2 = jnp.stack([pm['W2'], pe['W2'], pq['W2']])
    b2 = jnp.stack([pm['b2'], pe['b2'], pq['b2']]).reshape(3, 1, 64)
    sc1 = jnp.stack([pm['bn1_g'], pe['bn1_g'], pq['bn1_g']]).reshape(3, 1, 64) * inv
    be1 = jnp.stack([pm['bn1_b'], pe['bn1_b'], pq['bn1_b']]).reshape(3, 1, 64)
    w3 = jnp.stack([pm['W3'], pe['W3'], pq['W3']])
    u = _mid(y, agg1, b1, w2, b2, sc1, be1, w3)

    agg2 = _sc_segsum_24(src2d, dst2d, u, jnp.zeros((CHUNK, 24), _f32))

    b3 = jnp.concatenate([pm['b3'], pe['b3'], pq['b3']]).reshape(1, 24)
    w4 = jnp.stack([pm['W4'], pe['W4'], pq['W4']])
    b4 = jnp.stack([pm['b4'], pe['b4'], pq['b4']]).reshape(3, 1, 8)
    sc2 = jnp.stack([pm['bn2_g'], pe['bn2_g'], pq['bn2_g']]).reshape(3, 1, 8) * inv
    be2 = jnp.stack([pm['bn2_b'], pe['bn2_b'], pq['bn2_b']]).reshape(3, 1, 8)
    batch3d = batch_g1.reshape(NBLK, 1, BLK)

    aw = t['att_W'].reshape(8)
    z8 = jnp.zeros((8,), _f32)
    attw3 = jnp.stack(
        [jnp.concatenate([aw, z8, z8]), jnp.concatenate([z8, aw, z8]),
         jnp.concatenate([z8, z8, aw])] + [jnp.zeros((24,), _f32)] * 5, axis=1)
    attb8 = jnp.concatenate(
        [jnp.broadcast_to(t['att_b'].reshape(1, 1), (1, 3)),
         jnp.full((1, 5), -1e30, _f32)], axis=1)
    fc1w8 = jnp.concatenate([t['fc1_W'], jnp.zeros((7, 16), _f32)], axis=0)
    bcast8 = lambda v: jnp.broadcast_to(v.reshape(1, 1), (1, 8))

    sim, comp, embu = _tail(
        u, agg2, batch3d, b3, w4, b4, sc2, be2,
        attw3, attb8,
        fc1w8, bcast8(t['fc1_b']),
        bcast8(t['fc1_bn_g'] * inv), bcast8(t['fc1_bn_b']),
        t['fc2_W'], t['fc2_b'].reshape(1, 8),
        (t['fc2_bn_g'] * inv).reshape(1, 8), t['fc2_bn_b'].reshape(1, 8),
    )
    return (sim, comp, embu)
